# baseline (speedup 1.0000x reference)
import functools

import jax
import jax.numpy as jnp
from jax import lax
from jax.experimental import pallas as pl
from jax.experimental.pallas import tpu as pltpu
from jax.experimental.pallas import tpu_sc as plsc

L = 16
NC = 2
NS = 16
NW = NC * NS


@functools.lru_cache(maxsize=None)
def _make_sc_kernel(N: int, D: int, CH: int, NBUF: int):
    rows_per_w = N // NW
    n_chunks = rows_per_w // CH
    mesh = plsc.VectorSubcoreMesh(core_axis_name="c", subcore_axis_name="s")

    scratch = [pltpu.VMEM((D,), jnp.int32)]
    scratch += [pltpu.VMEM((CH, D), jnp.float32) for _ in range(2 * NBUF)]
    scratch += [pltpu.SemaphoreType.DMA for _ in range(2 * NBUF)]

    @functools.partial(
        pl.kernel,
        mesh=mesh,
        compiler_params=pltpu.CompilerParams(needs_layout_passes=False),
        out_type=jax.ShapeDtypeStruct((N, D), jnp.float32),
        scratch_types=scratch,
    )
    def k(x_hbm, idx_hbm, out_hbm, idx_v, *bufs_and_sems):
        ins = bufs_and_sems[:NBUF]
        outs = bufs_and_sems[NBUF:2 * NBUF]
        isems = bufs_and_sems[2 * NBUF:3 * NBUF]
        osems = bufs_and_sems[3 * NBUF:4 * NBUF]

        wid = lax.axis_index("s") * NC + lax.axis_index("c")
        base = wid * rows_per_w
        pltpu.sync_copy(idx_hbm, idx_v)

        def in_start(c, b):
            pltpu.async_copy(x_hbm.at[pl.ds(base + c * CH, CH)],
                             ins[b], isems[b])

        def in_wait(b):
            pltpu.make_async_copy(x_hbm.at[pl.ds(base, CH)],
                                  ins[b], isems[b]).wait()

        def out_start(c, b):
            pltpu.async_copy(outs[b],
                             out_hbm.at[pl.ds(base + c * CH, CH)], osems[b])

        def out_wait(b):
            pltpu.make_async_copy(outs[b],
                                  out_hbm.at[pl.ds(base, CH)], osems[b]).wait()

        for b in range(NBUF):
            in_start(b, b)
        n_rounds = (n_chunks + NBUF - 1) // NBUF

        def round_body(g, carry):
            for b in range(NBUF):
                c = g * NBUF + b

                @pl.when(c < n_chunks)
                def _():
                    in_wait(b)

                    @pl.when(g > 0)
                    def _():
                        out_wait(b)

                    @plsc.parallel_loop(0, D // L, 1, unroll=8)
                    def _(j):
                        idxv = idx_v[pl.ds(j * L, L)]
                        for r in range(CH):
                            rvec = jnp.full((L,), r, jnp.int32)
                            vals = plsc.load_gather(ins[b], [rvec, idxv])
                            outs[b][r, pl.ds(j * L, L)] = vals

                    out_start(c, b)

                    @pl.when(c + NBUF < n_chunks)
                    def _():
                        in_start(c + NBUF, b)
            return carry

        lax.fori_loop(0, n_rounds, round_body, 0)
        for b in range(NBUF):
            out_wait(b)

    return k


def kernel(x, indices):
    B, S, D = x.shape
    N = B * S
    k = _make_sc_kernel(N, D, 8, 3)
    out = k(x.reshape(N, D), indices)
    return out.reshape(B, S, D)


# speedup vs baseline: 1.0133x; 1.0133x over previous
import functools

import jax
import jax.numpy as jnp
from jax import lax
from jax.experimental import pallas as pl
from jax.experimental.pallas import tpu as pltpu
from jax.experimental.pallas import tpu_sc as plsc

L = 16
NC = 2
NS = 16
NW = NC * NS


@functools.lru_cache(maxsize=None)
def _make_sc_kernel(N: int, D: int, CH: int, NBUF: int):
    rows_per_w = N // NW
    n_chunks = rows_per_w // CH
    mesh = plsc.VectorSubcoreMesh(core_axis_name="c", subcore_axis_name="s")

    scratch = [pltpu.VMEM((D,), jnp.int32)]
    scratch += [pltpu.VMEM((CH, D), jnp.float32) for _ in range(2 * NBUF)]
    scratch += [pltpu.SemaphoreType.DMA for _ in range(2 * NBUF)]

    @functools.partial(
        pl.kernel,
        mesh=mesh,
        compiler_params=pltpu.CompilerParams(needs_layout_passes=False),
        out_type=jax.ShapeDtypeStruct((N, D), jnp.float32),
        scratch_types=scratch,
    )
    def k(x_hbm, idx_hbm, out_hbm, idx_v, *bufs_and_sems):
        ins = bufs_and_sems[:NBUF]
        outs = bufs_and_sems[NBUF:2 * NBUF]
        isems = bufs_and_sems[2 * NBUF:3 * NBUF]
        osems = bufs_and_sems[3 * NBUF:4 * NBUF]

        wid = lax.axis_index("s") * NC + lax.axis_index("c")
        base = wid * rows_per_w
        pltpu.sync_copy(idx_hbm, idx_v)

        def in_start(c, b):
            pltpu.async_copy(x_hbm.at[pl.ds(base + c * CH, CH)],
                             ins[b], isems[b])

        def in_wait(b):
            pltpu.make_async_copy(x_hbm.at[pl.ds(base, CH)],
                                  ins[b], isems[b]).wait()

        def out_start(c, b):
            pltpu.async_copy(outs[b],
                             out_hbm.at[pl.ds(base + c * CH, CH)], osems[b])

        def out_wait(b):
            pltpu.make_async_copy(outs[b],
                                  out_hbm.at[pl.ds(base, CH)], osems[b]).wait()

        for b in range(NBUF):
            in_start(b, b)
        n_rounds = (n_chunks + NBUF - 1) // NBUF

        def round_body(g, carry):
            for b in range(NBUF):
                c = g * NBUF + b

                @pl.when(c < n_chunks)
                def _():
                    in_wait(b)

                    @pl.when(g > 0)
                    def _():
                        out_wait(b)

                    @plsc.parallel_loop(0, D // L, 1, unroll=8)
                    def _(j):
                        idxv = idx_v[pl.ds(j * L, L)]
                        for r in range(CH):
                            rvec = jnp.full((L,), r, jnp.int32)
                            vals = plsc.load_gather(ins[b], [rvec, idxv])
                            outs[b][r, pl.ds(j * L, L)] = vals

                    out_start(c, b)

                    @pl.when(c + NBUF < n_chunks)
                    def _():
                        in_start(c + NBUF, b)
            return carry

        lax.fori_loop(0, n_rounds, round_body, 0)
        for b in range(NBUF):
            out_wait(b)

    return k


def kernel(x, indices):
    B, S, D = x.shape
    N = B * S
    k = _make_sc_kernel(N, D, 4, 4)
    out = k(x.reshape(N, D), indices)
    return out.reshape(B, S, D)
